# Initial kernel scaffold; baseline (speedup 1.0000x reference)
import jax
import jax.numpy as jnp
from jax.experimental import pallas as pl


def kernel(pred_mz, pred_intensity, pred_confidence, target_mz, target_intensity, target_mask):
    raise NotImplementedError("write your pallas kernel here")


# speedup vs baseline: 37.7581x; 37.7581x over previous
import jax
import jax.numpy as jnp
from jax import lax
from jax.experimental import pallas as pl
from jax.experimental.pallas import tpu as pltpu
from jax.experimental.pallas import tpu_sc as plsc

_B, _P, _T = 8192, 512, 512
_NBINS = 2000
_L = 16
_NC, _NS = 2, 16
_NW = _NC * _NS
_RPW = _B // _NW
_CHUNK = 8
_NCHUNK = _RPW // _CHUNK


def _sc_body(pmz_h, pint_h, pconf_h, tmz_h, tint_h, tmsk_h, out_h,
             bpm, bpi, bpc, btm, bti, btk,
             hp, ht, pidx, w_s, tidx, v_s, acc):
  wid = lax.axis_index("s") * _NC + lax.axis_index("c")
  row0 = wid * _RPW
  zero = jnp.zeros((_L,), jnp.float32)

  for k in range(_NBINS // _L):
    hp[pl.ds(k * _L, _L)] = zero
    ht[pl.ds(k * _L, _L)] = zero

  def chunk_body(c, carry):
    base = row0 + c * _CHUNK
    pltpu.sync_copy(pmz_h.at[pl.ds(base, _CHUNK)], bpm)
    pltpu.sync_copy(pint_h.at[pl.ds(base, _CHUNK)], bpi)
    pltpu.sync_copy(pconf_h.at[pl.ds(base, _CHUNK)], bpc)
    pltpu.sync_copy(tmz_h.at[pl.ds(base, _CHUNK)], btm)
    pltpu.sync_copy(tint_h.at[pl.ds(base, _CHUNK)], bti)
    pltpu.sync_copy(tmsk_h.at[pl.ds(base, _CHUNK)], btk)

    def row_body(r, carry2):
      for k in range(_P // _L):
        sl = pl.ds(k * _L, _L)
        idx = jnp.clip((bpm[r, sl] * 2000.0).astype(jnp.int32), 0, _NBINS - 1)
        wv = bpi[r, sl] * bpc[r, sl]
        pidx[sl] = idx
        w_s[sl] = wv
        plsc.addupdate_scatter(hp, [idx], wv)
      for k in range(_T // _L):
        sl = pl.ds(k * _L, _L)
        idx = jnp.clip((btm[r, sl] * 2000.0).astype(jnp.int32), 0, _NBINS - 1)
        vv = bti[r, sl] * btk[r, sl]
        tidx[sl] = idx
        v_s[sl] = vv
        plsc.addupdate_scatter(ht, [idx], vv)
      app = zero
      apt = zero
      att = zero
      for k in range(_P // _L):
        sl = pl.ds(k * _L, _L)
        idx = pidx[sl]
        wv = w_s[sl]
        app = app + wv * plsc.load_gather(hp, [idx])
        apt = apt + wv * plsc.load_gather(ht, [idx])
      for k in range(_T // _L):
        sl = pl.ds(k * _L, _L)
        att = att + v_s[sl] * plsc.load_gather(ht, [tidx[sl]])
      for k in range(_P // _L):
        plsc.store_scatter(hp, [pidx[pl.ds(k * _L, _L)]], zero)
      for k in range(_T // _L):
        plsc.store_scatter(ht, [tidx[pl.ds(k * _L, _L)]], zero)
      row = c * _CHUNK + r
      acc[row, 0:_L] = app
      acc[row, _L:2 * _L] = att
      acc[row, 2 * _L:3 * _L] = apt
      return carry2

    lax.fori_loop(0, _CHUNK, row_body, 0)
    return carry

  lax.fori_loop(0, _NCHUNK, chunk_body, 0)
  pltpu.sync_copy(acc, out_h.at[pl.ds(row0, _RPW)])


def _sc_hist_sums(pmz, pint, pconf, tmz, tint, tmsk):
  mesh = plsc.VectorSubcoreMesh(core_axis_name="c", subcore_axis_name="s",
                                num_cores=_NC, num_subcores=_NS)
  kfn = pl.kernel(
      _sc_body,
      out_type=jax.ShapeDtypeStruct((_B, 3 * _L), jnp.float32),
      mesh=mesh,
      scratch_types=[
          pltpu.VMEM((_CHUNK, _P), jnp.float32),
          pltpu.VMEM((_CHUNK, _P), jnp.float32),
          pltpu.VMEM((_CHUNK, _P), jnp.float32),
          pltpu.VMEM((_CHUNK, _T), jnp.float32),
          pltpu.VMEM((_CHUNK, _T), jnp.float32),
          pltpu.VMEM((_CHUNK, _T), jnp.float32),
          pltpu.VMEM((_NBINS,), jnp.float32),
          pltpu.VMEM((_NBINS,), jnp.float32),
          pltpu.VMEM((_P,), jnp.int32),
          pltpu.VMEM((_P,), jnp.float32),
          pltpu.VMEM((_T,), jnp.int32),
          pltpu.VMEM((_T,), jnp.float32),
          pltpu.VMEM((_RPW, 3 * _L), jnp.float32),
      ],
      compiler_params=pltpu.CompilerParams(needs_layout_passes=False),
  )
  return kfn(pmz, pint, pconf, tmz, tint, tmsk)


def _tc_finish_body(x_ref, o_ref):
  x = x_ref[...]
  pp = jnp.sum(x[:, 0:_L], axis=1, keepdims=True)
  tt = jnp.sum(x[:, _L:2 * _L], axis=1, keepdims=True)
  pt = jnp.sum(x[:, 2 * _L:3 * _L], axis=1, keepdims=True)
  a = jnp.sqrt(pp)
  b = jnp.sqrt(tt)
  ae = a + 1e-8
  be = b + 1e-8
  pn = jnp.maximum(a / ae, 1e-8)
  tn = jnp.maximum(b / be, 1e-8)
  cos = (pt / (ae * be)) / (pn * tn)
  o_ref[0, 0] = 1.0 - jnp.mean(cos)


def _tc_finish(sums):
  return pl.pallas_call(
      _tc_finish_body,
      out_shape=jax.ShapeDtypeStruct((1, 1), jnp.float32),
      out_specs=pl.BlockSpec(memory_space=pltpu.SMEM),
  )(sums)


@jax.jit
def kernel(pred_mz, pred_intensity, pred_confidence,
           target_mz, target_intensity, target_mask):
  sums = _sc_hist_sums(pred_mz, pred_intensity, pred_confidence,
                       target_mz, target_intensity, target_mask)
  return _tc_finish(sums)[0, 0]


# speedup vs baseline: 51.0028x; 1.3508x over previous
import jax
import jax.numpy as jnp
from jax import lax
from jax.experimental import pallas as pl
from jax.experimental.pallas import tpu as pltpu
from jax.experimental.pallas import tpu_sc as plsc

_B, _P, _T = 8192, 512, 512
_NBINS = 2000
_L = 16
_NC, _NS = 2, 16
_NW = _NC * _NS
_RPW = _B // _NW
_CHUNK = 8
_NCHUNK = _RPW // _CHUNK


def _sc_body(pmz_h, pint_h, pconf_h, tmz_h, tint_h, tmsk_h, out_h,
             bpm, bpi, bpc, btm, bti, btk,
             hp, ht, pidx, w_s, tidx, v_s, acc, sem0, sem1):
  wid = lax.axis_index("s") * _NC + lax.axis_index("c")
  row0 = wid * _RPW
  zero = jnp.zeros((_L,), jnp.float32)

  for k in range(_NBINS // _L):
    hp[pl.ds(k * _L, _L)] = zero
    ht[pl.ds(k * _L, _L)] = zero

  srcs = (pmz_h, pint_h, pconf_h, tmz_h, tint_h, tmsk_h)
  bufs = (bpm, bpi, bpc, btm, bti, btk)

  def issue(c, slot, sem):
    base = row0 + c * _CHUNK
    for src, buf in zip(srcs, bufs):
      pltpu.async_copy(src.at[pl.ds(base, _CHUNK)], buf.at[slot], sem)

  def drain(c, slot, sem):
    base = row0 + c * _CHUNK
    for src, buf in zip(srcs, bufs):
      pltpu.make_async_copy(src.at[pl.ds(base, _CHUNK)], buf.at[slot], sem).wait()

  def process(c, slot):
    lbpm, lbpi, lbpc = bpm.at[slot], bpi.at[slot], bpc.at[slot]
    lbtm, lbti, lbtk = btm.at[slot], bti.at[slot], btk.at[slot]

    def row_body(r, carry2):
      for k in range(_P // _L):
        sl = pl.ds(k * _L, _L)
        idx = jnp.clip((lbpm[r, sl] * 2000.0).astype(jnp.int32), 0, _NBINS - 1)
        wv = lbpi[r, sl] * lbpc[r, sl]
        pidx[sl] = idx
        w_s[sl] = wv
        plsc.addupdate_scatter(hp, [idx], wv)
      for k in range(_T // _L):
        sl = pl.ds(k * _L, _L)
        idx = jnp.clip((lbtm[r, sl] * 2000.0).astype(jnp.int32), 0, _NBINS - 1)
        vv = lbti[r, sl] * lbtk[r, sl]
        tidx[sl] = idx
        v_s[sl] = vv
        plsc.addupdate_scatter(ht, [idx], vv)
      app = zero
      apt = zero
      att = zero
      for k in range(_P // _L):
        sl = pl.ds(k * _L, _L)
        idx = pidx[sl]
        wv = w_s[sl]
        app = app + wv * plsc.load_gather(hp, [idx])
        apt = apt + wv * plsc.load_gather(ht, [idx])
      for k in range(_T // _L):
        sl = pl.ds(k * _L, _L)
        att = att + v_s[sl] * plsc.load_gather(ht, [tidx[sl]])
      for k in range(_P // _L):
        plsc.store_scatter(hp, [pidx[pl.ds(k * _L, _L)]], zero)
      for k in range(_T // _L):
        plsc.store_scatter(ht, [tidx[pl.ds(k * _L, _L)]], zero)
      row = c * _CHUNK + r
      acc[row, 0:_L] = app
      acc[row, _L:2 * _L] = att
      acc[row, 2 * _L:3 * _L] = apt
      return carry2

    lax.fori_loop(0, _CHUNK, row_body, 0)

  npair = _NCHUNK // 2
  issue(0, 0, sem0)

  def pair_body(c2, carry):
    c0 = 2 * c2
    c1 = c0 + 1
    issue(c1, 1, sem1)
    drain(c0, 0, sem0)
    process(c0, 0)

    @pl.when(c2 < npair - 1)
    def _prefetch():
      issue(c0 + 2, 0, sem0)

    drain(c1, 1, sem1)
    process(c1, 1)
    return carry

  lax.fori_loop(0, npair, pair_body, 0)
  pltpu.sync_copy(acc, out_h.at[pl.ds(row0, _RPW)])


def _sc_hist_sums(pmz, pint, pconf, tmz, tint, tmsk):
  mesh = plsc.VectorSubcoreMesh(core_axis_name="c", subcore_axis_name="s",
                                num_cores=_NC, num_subcores=_NS)
  kfn = pl.kernel(
      _sc_body,
      out_type=jax.ShapeDtypeStruct((_B, 3 * _L), jnp.float32),
      mesh=mesh,
      scratch_types=[
          pltpu.VMEM((2, _CHUNK, _P), jnp.float32),
          pltpu.VMEM((2, _CHUNK, _P), jnp.float32),
          pltpu.VMEM((2, _CHUNK, _P), jnp.float32),
          pltpu.VMEM((2, _CHUNK, _T), jnp.float32),
          pltpu.VMEM((2, _CHUNK, _T), jnp.float32),
          pltpu.VMEM((2, _CHUNK, _T), jnp.float32),
          pltpu.VMEM((_NBINS,), jnp.float32),
          pltpu.VMEM((_NBINS,), jnp.float32),
          pltpu.VMEM((_P,), jnp.int32),
          pltpu.VMEM((_P,), jnp.float32),
          pltpu.VMEM((_T,), jnp.int32),
          pltpu.VMEM((_T,), jnp.float32),
          pltpu.VMEM((_RPW, 3 * _L), jnp.float32),
          pltpu.SemaphoreType.DMA,
          pltpu.SemaphoreType.DMA,
      ],
      compiler_params=pltpu.CompilerParams(needs_layout_passes=False),
  )
  return kfn(pmz, pint, pconf, tmz, tint, tmsk)


def _tc_finish_body(x_ref, o_ref):
  x = x_ref[...]
  pp = jnp.sum(x[:, 0:_L], axis=1, keepdims=True)
  tt = jnp.sum(x[:, _L:2 * _L], axis=1, keepdims=True)
  pt = jnp.sum(x[:, 2 * _L:3 * _L], axis=1, keepdims=True)
  a = jnp.sqrt(pp)
  b = jnp.sqrt(tt)
  ae = a + 1e-8
  be = b + 1e-8
  pn = jnp.maximum(a / ae, 1e-8)
  tn = jnp.maximum(b / be, 1e-8)
  cos = (pt / (ae * be)) / (pn * tn)
  o_ref[0, 0] = 1.0 - jnp.mean(cos)


def _tc_finish(sums):
  return pl.pallas_call(
      _tc_finish_body,
      out_shape=jax.ShapeDtypeStruct((1, 1), jnp.float32),
      out_specs=pl.BlockSpec(memory_space=pltpu.SMEM),
  )(sums)


@jax.jit
def kernel(pred_mz, pred_intensity, pred_confidence,
           target_mz, target_intensity, target_mask):
  sums = _sc_hist_sums(pred_mz, pred_intensity, pred_confidence,
                       target_mz, target_intensity, target_mask)
  return _tc_finish(sums)[0, 0]


# speedup vs baseline: 69.4537x; 1.3618x over previous
import jax
import jax.numpy as jnp
from jax import lax
from jax.experimental import pallas as pl
from jax.experimental.pallas import tpu as pltpu
from jax.experimental.pallas import tpu_sc as plsc

_B, _P, _T = 8192, 512, 512
_NBINS = 2000
_L = 16
_NC, _NS = 2, 16
_NW = _NC * _NS
_RPW = _B // _NW
_CHUNK = 8
_NCHUNK = _RPW // _CHUNK


def _sc_body(pmz_h, pint_h, pconf_h, tmz_h, tint_h, tmsk_h, out_h,
             bpm, bpi, bpc, btm, bti, btk,
             hp, ht, acc, sem0, sem1):
  wid = lax.axis_index("s") * _NC + lax.axis_index("c")
  row0 = wid * _RPW
  zero = jnp.zeros((_L,), jnp.float32)

  for k in range(_NBINS // _L):
    hp[pl.ds(k * _L, _L)] = zero
    ht[pl.ds(k * _L, _L)] = zero

  srcs = (pmz_h, pint_h, pconf_h, tmz_h, tint_h, tmsk_h)
  bufs = (bpm, bpi, bpc, btm, bti, btk)

  def issue(c, slot, sem):
    base = row0 + c * _CHUNK
    for src, buf in zip(srcs, bufs):
      pltpu.async_copy(src.at[pl.ds(base, _CHUNK)], buf.at[slot], sem)

  def drain(c, slot, sem):
    base = row0 + c * _CHUNK
    for src, buf in zip(srcs, bufs):
      pltpu.make_async_copy(src.at[pl.ds(base, _CHUNK)], buf.at[slot], sem).wait()

  def process(c, slot):
    lbpm, lbpi, lbpc = bpm.at[slot], bpi.at[slot], bpc.at[slot]
    lbtm, lbti, lbtk = btm.at[slot], bti.at[slot], btk.at[slot]

    def row_body(r, carry2):
      for k in range(_P // _L):
        sl = pl.ds(k * _L, _L)
        idx = jnp.clip((lbpm[r, sl] * 2000.0).astype(jnp.int32), 0, _NBINS - 1)
        wv = lbpi[r, sl] * lbpc[r, sl]
        plsc.addupdate_scatter(hp, [idx], wv)
      for k in range(_T // _L):
        sl = pl.ds(k * _L, _L)
        idx = jnp.clip((lbtm[r, sl] * 2000.0).astype(jnp.int32), 0, _NBINS - 1)
        vv = lbti[r, sl] * lbtk[r, sl]
        plsc.addupdate_scatter(ht, [idx], vv)
      napp = [zero, zero]
      natt = [zero, zero]
      napt = [zero, zero]
      for k in range(_NBINS // _L):
        sl = pl.ds(k * _L, _L)
        hv = hp[sl]
        tv = ht[sl]
        j = k & 1
        napp[j] = napp[j] + hv * hv
        natt[j] = natt[j] + tv * tv
        napt[j] = napt[j] + hv * tv
        hp[sl] = zero
        ht[sl] = zero
      row = c * _CHUNK + r
      acc[row, 0:_L] = napp[0] + napp[1]
      acc[row, _L:2 * _L] = natt[0] + natt[1]
      acc[row, 2 * _L:3 * _L] = napt[0] + napt[1]
      return carry2

    lax.fori_loop(0, _CHUNK, row_body, 0)

  npair = _NCHUNK // 2
  issue(0, 0, sem0)

  def pair_body(c2, carry):
    c0 = 2 * c2
    c1 = c0 + 1
    issue(c1, 1, sem1)
    drain(c0, 0, sem0)
    process(c0, 0)

    @pl.when(c2 < npair - 1)
    def _prefetch():
      issue(c0 + 2, 0, sem0)

    drain(c1, 1, sem1)
    process(c1, 1)
    return carry

  lax.fori_loop(0, npair, pair_body, 0)
  pltpu.sync_copy(acc, out_h.at[pl.ds(row0, _RPW)])


def _sc_hist_sums(pmz, pint, pconf, tmz, tint, tmsk):
  mesh = plsc.VectorSubcoreMesh(core_axis_name="c", subcore_axis_name="s",
                                num_cores=_NC, num_subcores=_NS)
  kfn = pl.kernel(
      _sc_body,
      out_type=jax.ShapeDtypeStruct((_B, 3 * _L), jnp.float32),
      mesh=mesh,
      scratch_types=[
          pltpu.VMEM((2, _CHUNK, _P), jnp.float32),
          pltpu.VMEM((2, _CHUNK, _P), jnp.float32),
          pltpu.VMEM((2, _CHUNK, _P), jnp.float32),
          pltpu.VMEM((2, _CHUNK, _T), jnp.float32),
          pltpu.VMEM((2, _CHUNK, _T), jnp.float32),
          pltpu.VMEM((2, _CHUNK, _T), jnp.float32),
          pltpu.VMEM((_NBINS,), jnp.float32),
          pltpu.VMEM((_NBINS,), jnp.float32),
          pltpu.VMEM((_RPW, 3 * _L), jnp.float32),
          pltpu.SemaphoreType.DMA,
          pltpu.SemaphoreType.DMA,
      ],
      compiler_params=pltpu.CompilerParams(needs_layout_passes=False),
  )
  return kfn(pmz, pint, pconf, tmz, tint, tmsk)


def _tc_finish_body(x_ref, o_ref):
  x = x_ref[...]
  pp = jnp.sum(x[:, 0:_L], axis=1, keepdims=True)
  tt = jnp.sum(x[:, _L:2 * _L], axis=1, keepdims=True)
  pt = jnp.sum(x[:, 2 * _L:3 * _L], axis=1, keepdims=True)
  a = jnp.sqrt(pp)
  b = jnp.sqrt(tt)
  ae = a + 1e-8
  be = b + 1e-8
  pn = jnp.maximum(a / ae, 1e-8)
  tn = jnp.maximum(b / be, 1e-8)
  cos = (pt / (ae * be)) / (pn * tn)
  o_ref[0, 0] = 1.0 - jnp.mean(cos)


def _tc_finish(sums):
  return pl.pallas_call(
      _tc_finish_body,
      out_shape=jax.ShapeDtypeStruct((1, 1), jnp.float32),
      out_specs=pl.BlockSpec(memory_space=pltpu.SMEM),
  )(sums)


@jax.jit
def kernel(pred_mz, pred_intensity, pred_confidence,
           target_mz, target_intensity, target_mask):
  sums = _sc_hist_sums(pred_mz, pred_intensity, pred_confidence,
                       target_mz, target_intensity, target_mask)
  return _tc_finish(sums)[0, 0]


# speedup vs baseline: 141.3497x; 2.0352x over previous
import jax
import jax.numpy as jnp
from jax import lax
from jax.experimental import pallas as pl
from jax.experimental.pallas import tpu as pltpu
from jax.experimental.pallas import tpu_sc as plsc

_B, _P, _T = 8192, 512, 512
_NBINS = 2000
_L = 16
_NC, _NS = 2, 16
_NW = _NC * _NS
_RPW = _B // _NW
_CHUNK = 8
_NCHUNK = _RPW // _CHUNK


def _sc_body(pmz_h, pint_h, pconf_h, tmz_h, tint_h, tmsk_h, out_h,
             bpm, bpi, bpc, btm, bti, btk,
             hp, ht, acc, sem0, sem1):
  wid = lax.axis_index("s") * _NC + lax.axis_index("c")
  row0 = wid * _RPW
  zero = jnp.zeros((_L,), jnp.float32)

  for k in range(_NBINS // _L):
    hp[pl.ds(k * _L, _L)] = zero
    ht[pl.ds(k * _L, _L)] = zero

  srcs = (pmz_h, pint_h, pconf_h, tmz_h, tint_h, tmsk_h)
  bufs = (bpm, bpi, bpc, btm, bti, btk)

  def issue(c, slot, sem):
    base = row0 + c * _CHUNK
    for src, buf in zip(srcs, bufs):
      pltpu.async_copy(src.at[pl.ds(base, _CHUNK)], buf.at[slot], sem)

  def drain(c, slot, sem):
    base = row0 + c * _CHUNK
    for src, buf in zip(srcs, bufs):
      pltpu.make_async_copy(src.at[pl.ds(base, _CHUNK)], buf.at[slot], sem).wait()

  def process(c, slot):
    lbpm, lbpi, lbpc = bpm.at[slot], bpi.at[slot], bpc.at[slot]
    lbtm, lbti, lbtk = btm.at[slot], bti.at[slot], btk.at[slot]

    def row_body(r, carry2):
      @plsc.parallel_loop(0, _P, step=_L, unroll=4)
      def _build_p(i):
        sl = pl.ds(i, _L)
        idx = jnp.clip((lbpm[r, sl] * 2000.0).astype(jnp.int32), 0, _NBINS - 1)
        wv = lbpi[r, sl] * lbpc[r, sl]
        plsc.addupdate_scatter(hp, [idx], wv)

      @plsc.parallel_loop(0, _T, step=_L, unroll=4)
      def _build_t(i):
        sl = pl.ds(i, _L)
        idx = jnp.clip((lbtm[r, sl] * 2000.0).astype(jnp.int32), 0, _NBINS - 1)
        vv = lbti[r, sl] * lbtk[r, sl]
        plsc.addupdate_scatter(ht, [idx], vv)

      def _sweep(i, accs):
        app, att, apt = accs
        sl = pl.ds(i, _L)
        hv = hp[sl]
        tv = ht[sl]
        app = app + hv * hv
        att = att + tv * tv
        apt = apt + hv * tv
        hp[sl] = zero
        ht[sl] = zero
        return (app, att, apt)

      app, att, apt = plsc.parallel_loop(
          0, _NBINS, step=_L, unroll=4, carry=(zero, zero, zero))(_sweep)
      row = c * _CHUNK + r
      acc[row, 0:_L] = app
      acc[row, _L:2 * _L] = att
      acc[row, 2 * _L:3 * _L] = apt
      return carry2

    lax.fori_loop(0, _CHUNK, row_body, 0)

  npair = _NCHUNK // 2
  issue(0, 0, sem0)

  def pair_body(c2, carry):
    c0 = 2 * c2
    c1 = c0 + 1
    issue(c1, 1, sem1)
    drain(c0, 0, sem0)
    process(c0, 0)

    @pl.when(c2 < npair - 1)
    def _prefetch():
      issue(c0 + 2, 0, sem0)

    drain(c1, 1, sem1)
    process(c1, 1)
    return carry

  lax.fori_loop(0, npair, pair_body, 0)
  pltpu.sync_copy(acc, out_h.at[pl.ds(row0, _RPW)])


def _sc_hist_sums(pmz, pint, pconf, tmz, tint, tmsk):
  mesh = plsc.VectorSubcoreMesh(core_axis_name="c", subcore_axis_name="s",
                                num_cores=_NC, num_subcores=_NS)
  kfn = pl.kernel(
      _sc_body,
      out_type=jax.ShapeDtypeStruct((_B, 3 * _L), jnp.float32),
      mesh=mesh,
      scratch_types=[
          pltpu.VMEM((2, _CHUNK, _P), jnp.float32),
          pltpu.VMEM((2, _CHUNK, _P), jnp.float32),
          pltpu.VMEM((2, _CHUNK, _P), jnp.float32),
          pltpu.VMEM((2, _CHUNK, _T), jnp.float32),
          pltpu.VMEM((2, _CHUNK, _T), jnp.float32),
          pltpu.VMEM((2, _CHUNK, _T), jnp.float32),
          pltpu.VMEM((_NBINS,), jnp.float32),
          pltpu.VMEM((_NBINS,), jnp.float32),
          pltpu.VMEM((_RPW, 3 * _L), jnp.float32),
          pltpu.SemaphoreType.DMA,
          pltpu.SemaphoreType.DMA,
      ],
      compiler_params=pltpu.CompilerParams(needs_layout_passes=False),
  )
  return kfn(pmz, pint, pconf, tmz, tint, tmsk)


def _tc_finish_body(x_ref, o_ref):
  x = x_ref[...]
  pp = jnp.sum(x[:, 0:_L], axis=1, keepdims=True)
  tt = jnp.sum(x[:, _L:2 * _L], axis=1, keepdims=True)
  pt = jnp.sum(x[:, 2 * _L:3 * _L], axis=1, keepdims=True)
  a = jnp.sqrt(pp)
  b = jnp.sqrt(tt)
  ae = a + 1e-8
  be = b + 1e-8
  pn = jnp.maximum(a / ae, 1e-8)
  tn = jnp.maximum(b / be, 1e-8)
  cos = (pt / (ae * be)) / (pn * tn)
  o_ref[0, 0] = 1.0 - jnp.mean(cos)


def _tc_finish(sums):
  return pl.pallas_call(
      _tc_finish_body,
      out_shape=jax.ShapeDtypeStruct((1, 1), jnp.float32),
      out_specs=pl.BlockSpec(memory_space=pltpu.SMEM),
  )(sums)


@jax.jit
def kernel(pred_mz, pred_intensity, pred_confidence,
           target_mz, target_intensity, target_mask):
  sums = _sc_hist_sums(pred_mz, pred_intensity, pred_confidence,
                       target_mz, target_intensity, target_mask)
  return _tc_finish(sums)[0, 0]


# speedup vs baseline: 143.4588x; 1.0149x over previous
import jax
import jax.numpy as jnp
from jax import lax
from jax.experimental import pallas as pl
from jax.experimental.pallas import tpu as pltpu
from jax.experimental.pallas import tpu_sc as plsc

_B, _P, _T = 8192, 512, 512
_NBINS = 2000
_L = 16
_NC, _NS = 2, 16
_NW = _NC * _NS
_RPW = _B // _NW
_CHUNK = 8
_NCHUNK = _RPW // _CHUNK


def _sc_body(pmz_h, pint_h, pconf_h, tmz_h, tint_h, tmsk_h, out_h,
             bpm, bpi, bpc, btm, bti, btk,
             hp, ht, acc, sem0, sem1):
  wid = lax.axis_index("s") * _NC + lax.axis_index("c")
  row0 = wid * _RPW
  zero = jnp.zeros((_L,), jnp.float32)

  for k in range(_NBINS // _L):
    hp[pl.ds(k * _L, _L)] = zero
    ht[pl.ds(k * _L, _L)] = zero

  srcs = (pmz_h, pint_h, pconf_h, tmz_h, tint_h, tmsk_h)
  bufs = (bpm, bpi, bpc, btm, bti, btk)

  def issue(c, slot, sem):
    base = row0 + c * _CHUNK
    for src, buf in zip(srcs, bufs):
      pltpu.async_copy(src.at[pl.ds(base, _CHUNK)], buf.at[slot], sem)

  def drain(c, slot, sem):
    base = row0 + c * _CHUNK
    for src, buf in zip(srcs, bufs):
      pltpu.make_async_copy(src.at[pl.ds(base, _CHUNK)], buf.at[slot], sem).wait()

  def process(c, slot):
    lbpm, lbpi, lbpc = bpm.at[slot], bpi.at[slot], bpc.at[slot]
    lbtm, lbti, lbtk = btm.at[slot], bti.at[slot], btk.at[slot]

    def row_body(r, carry2):
      @plsc.parallel_loop(0, _P, step=_L, unroll=8)
      def _build_p(i):
        sl = pl.ds(i, _L)
        idx = jnp.clip((lbpm[r, sl] * 2000.0).astype(jnp.int32), 0, _NBINS - 1)
        wv = lbpi[r, sl] * lbpc[r, sl]
        plsc.addupdate_scatter(hp, [idx], wv)

      @plsc.parallel_loop(0, _T, step=_L, unroll=8)
      def _build_t(i):
        sl = pl.ds(i, _L)
        idx = jnp.clip((lbtm[r, sl] * 2000.0).astype(jnp.int32), 0, _NBINS - 1)
        vv = lbti[r, sl] * lbtk[r, sl]
        plsc.addupdate_scatter(ht, [idx], vv)

      def _sweep(i, accs):
        app, att, apt = accs
        sl = pl.ds(i, _L)
        hv = hp[sl]
        tv = ht[sl]
        app = app + hv * hv
        att = att + tv * tv
        apt = apt + hv * tv
        hp[sl] = zero
        ht[sl] = zero
        return (app, att, apt)

      app, att, apt = plsc.parallel_loop(
          0, _NBINS, step=_L, unroll=8, carry=(zero, zero, zero))(_sweep)
      row = c * _CHUNK + r
      acc[row, 0:_L] = app
      acc[row, _L:2 * _L] = att
      acc[row, 2 * _L:3 * _L] = apt
      return carry2

    lax.fori_loop(0, _CHUNK, row_body, 0)

  npair = _NCHUNK // 2
  issue(0, 0, sem0)

  def pair_body(c2, carry):
    c0 = 2 * c2
    c1 = c0 + 1
    issue(c1, 1, sem1)
    drain(c0, 0, sem0)
    process(c0, 0)

    @pl.when(c2 < npair - 1)
    def _prefetch():
      issue(c0 + 2, 0, sem0)

    drain(c1, 1, sem1)
    process(c1, 1)
    return carry

  lax.fori_loop(0, npair, pair_body, 0)
  pltpu.sync_copy(acc, out_h.at[pl.ds(row0, _RPW)])


def _sc_hist_sums(pmz, pint, pconf, tmz, tint, tmsk):
  mesh = plsc.VectorSubcoreMesh(core_axis_name="c", subcore_axis_name="s",
                                num_cores=_NC, num_subcores=_NS)
  kfn = pl.kernel(
      _sc_body,
      out_type=jax.ShapeDtypeStruct((_B, 3 * _L), jnp.float32),
      mesh=mesh,
      scratch_types=[
          pltpu.VMEM((2, _CHUNK, _P), jnp.float32),
          pltpu.VMEM((2, _CHUNK, _P), jnp.float32),
          pltpu.VMEM((2, _CHUNK, _P), jnp.float32),
          pltpu.VMEM((2, _CHUNK, _T), jnp.float32),
          pltpu.VMEM((2, _CHUNK, _T), jnp.float32),
          pltpu.VMEM((2, _CHUNK, _T), jnp.float32),
          pltpu.VMEM((_NBINS,), jnp.float32),
          pltpu.VMEM((_NBINS,), jnp.float32),
          pltpu.VMEM((_RPW, 3 * _L), jnp.float32),
          pltpu.SemaphoreType.DMA,
          pltpu.SemaphoreType.DMA,
      ],
      compiler_params=pltpu.CompilerParams(needs_layout_passes=False),
  )
  return kfn(pmz, pint, pconf, tmz, tint, tmsk)


def _tc_finish_body(x_ref, o_ref):
  x = x_ref[...]
  pp = jnp.sum(x[:, 0:_L], axis=1, keepdims=True)
  tt = jnp.sum(x[:, _L:2 * _L], axis=1, keepdims=True)
  pt = jnp.sum(x[:, 2 * _L:3 * _L], axis=1, keepdims=True)
  a = jnp.sqrt(pp)
  b = jnp.sqrt(tt)
  ae = a + 1e-8
  be = b + 1e-8
  pn = jnp.maximum(a / ae, 1e-8)
  tn = jnp.maximum(b / be, 1e-8)
  cos = (pt / (ae * be)) / (pn * tn)
  o_ref[0, 0] = 1.0 - jnp.mean(cos)


def _tc_finish(sums):
  return pl.pallas_call(
      _tc_finish_body,
      out_shape=jax.ShapeDtypeStruct((1, 1), jnp.float32),
      out_specs=pl.BlockSpec(memory_space=pltpu.SMEM),
  )(sums)


@jax.jit
def kernel(pred_mz, pred_intensity, pred_confidence,
           target_mz, target_intensity, target_mask):
  sums = _sc_hist_sums(pred_mz, pred_intensity, pred_confidence,
                       target_mz, target_intensity, target_mask)
  return _tc_finish(sums)[0, 0]
